# baseline (speedup 1.0000x reference)
import functools

import jax
import jax.numpy as jnp
from jax import lax
from jax.experimental import pallas as pl
from jax.experimental.pallas import tpu as pltpu
from jax.experimental.pallas import tpu_sc as plsc

B = 16384
F = 26
VOCAB = 1000
ED = 128
H1 = 128
H2 = 64
EPS = 1e-5

ROWS_PER_CHUNK = 4
IDX_PER_CHUNK = ROWS_PER_CHUNK * F
LANES = 16


def _fold_body(t_ref, w_ref, p_ref):
    p_ref[...] = jnp.dot(t_ref[0], w_ref[...],
                         preferred_element_type=jnp.float32)[None]


_fold = pl.pallas_call(
    _fold_body,
    grid=(F,),
    in_specs=[
        pl.BlockSpec((1, VOCAB, ED), lambda f: (f, 0, 0)),
        pl.BlockSpec((ED, H1), lambda f: (f, 0)),
    ],
    out_specs=pl.BlockSpec((1, VOCAB, H1), lambda f: (f, 0, 0)),
    out_shape=jax.ShapeDtypeStruct((F, VOCAB, H1), jnp.float32),
)


NBUF = 4


def _make_bag(num_cores, num_subcores):
    nw = num_cores * num_subcores
    bpw = B // nw
    nchunks = bpw // ROWS_PER_CHUNK
    mesh = plsc.VectorSubcoreMesh(core_axis_name="c", subcore_axis_name="s")

    @functools.partial(
        pl.kernel,
        mesh=mesh,
        out_type=jax.ShapeDtypeStruct((B, H1), jnp.float32),
        scratch_types=[
            pltpu.VMEM((nchunks, IDX_PER_CHUNK), jnp.int32),
        ]
        + [pltpu.VMEM((IDX_PER_CHUNK, H1), jnp.float32) for _ in range(NBUF)]
        + [pltpu.VMEM((ROWS_PER_CHUNK, H1), jnp.float32) for _ in range(NBUF)]
        + [pltpu.SemaphoreType.DMA for _ in range(2 * NBUF)],
    )
    def bag(idx_hbm, p_hbm, out_hbm, idx_v, *rest):
        g_v = rest[:NBUF]
        o_v = rest[NBUF:2 * NBUF]
        sems = rest[2 * NBUF:3 * NBUF]
        osems = rest[3 * NBUF:]
        wid = lax.axis_index("s") * num_cores + lax.axis_index("c")
        base = wid * bpw
        pltpu.sync_copy(idx_hbm.at[wid], idx_v)

        for b in range(NBUF):
            pltpu.async_copy(p_hbm.at[idx_v.at[b]], g_v[b], sems[b])

        def outer(c0, carry):
            for b in range(NBUF):
                c = c0 + b
                pltpu.make_async_copy(p_hbm.at[idx_v.at[c]],
                                      g_v[b], sems[b]).wait()

                @pl.when(c >= NBUF)
                def _():
                    pltpu.make_async_copy(
                        o_v[b], out_hbm.at[pl.ds(base, ROWS_PER_CHUNK)],
                        osems[b]).wait()

                for r in range(ROWS_PER_CHUNK):
                    for v in range(H1 // LANES):
                        sl = pl.ds(v * LANES, LANES)
                        o_v[b][r, sl] = g_v[b][r * F, sl]
                    for f in range(1, F):
                        for v in range(H1 // LANES):
                            sl = pl.ds(v * LANES, LANES)
                            plsc.addupdate(o_v[b].at[r, sl],
                                           g_v[b][r * F + f, sl])

                @pl.when(c + NBUF < nchunks)
                def _():
                    pltpu.async_copy(p_hbm.at[idx_v.at[c + NBUF]],
                                     g_v[b], sems[b])

                pltpu.async_copy(
                    o_v[b],
                    out_hbm.at[pl.ds(base + c * ROWS_PER_CHUNK,
                                     ROWS_PER_CHUNK)],
                    osems[b])
            return carry

        lax.fori_loop(0, nchunks // NBUF, lambda i, cr: outer(i * NBUF, cr),
                      0)
        for b in range(NBUF):
            pltpu.make_async_copy(
                o_v[b], out_hbm.at[pl.ds(base, ROWS_PER_CHUNK)],
                osems[b]).wait()

    return bag


_TAIL_CHUNKS = 16
_TAIL_ROWS = B // _TAIL_CHUNKS


def _tail_body(h_ref, g_ref, be_ref, w2_ref, b2_ref, w3_ref, b3_ref,
               o_ref, sum_ref, sq_ref):
    p = pl.program_id(0)
    j = pl.program_id(1)

    @pl.when(jnp.logical_and(p == 0, j == 0))
    def _():
        sum_ref[...] = jnp.zeros_like(sum_ref)
        sq_ref[...] = jnp.zeros_like(sq_ref)

    @pl.when(p == 0)
    def _():
        h = h_ref[...]
        sum_ref[...] += jnp.sum(h, axis=0, keepdims=True)
        sq_ref[...] += jnp.sum(h * h, axis=0, keepdims=True)

    @pl.when(p == 1)
    def _():
        h = h_ref[...]
        mean = sum_ref[...] * (1.0 / B)
        var = sq_ref[...] * (1.0 / B) - mean * mean
        hn = g_ref[...] * (h - mean) * lax.rsqrt(var + EPS) + be_ref[...]
        h1 = jnp.maximum(hn, 0.0)
        h2 = jnp.maximum(
            jnp.dot(h1, w2_ref[...], preferred_element_type=jnp.float32)
            + b2_ref[...], 0.0)
        o_ref[...] = (jnp.dot(h2, w3_ref[...],
                              preferred_element_type=jnp.float32)
                      + b3_ref[...])


_tail = pl.pallas_call(
    _tail_body,
    grid=(2, _TAIL_CHUNKS),
    in_specs=[
        pl.BlockSpec((_TAIL_ROWS, H1), lambda p, j: (j, 0)),
        pl.BlockSpec((1, H1), lambda p, j: (0, 0)),
        pl.BlockSpec((1, H1), lambda p, j: (0, 0)),
        pl.BlockSpec((H1, H2), lambda p, j: (0, 0)),
        pl.BlockSpec((1, H2), lambda p, j: (0, 0)),
        pl.BlockSpec((H2, 1), lambda p, j: (0, 0)),
        pl.BlockSpec((1, 1), lambda p, j: (0, 0)),
    ],
    out_specs=pl.BlockSpec((_TAIL_ROWS, 1), lambda p, j: (j, 0)),
    out_shape=jax.ShapeDtypeStruct((B, 1), jnp.float32),
    scratch_shapes=[
        pltpu.VMEM((1, H1), jnp.float32),
        pltpu.VMEM((1, H1), jnp.float32),
    ],
)


def kernel(x, tables, W1, b1, gamma, beta, W2, b2, W3, b3):
    del b1
    P = _fold(tables, W1)

    info = plsc.get_sparse_core_info()
    nw = info.num_cores * info.num_subcores
    bpw = B // nw
    nchunks = bpw // ROWS_PER_CHUNK
    flat_idx = (x.astype(jnp.int32)
                + (jnp.arange(F, dtype=jnp.int32) * VOCAB)[None, :])
    flat_idx = flat_idx.reshape(nw, nchunks, IDX_PER_CHUNK)

    bag = _make_bag(info.num_cores, info.num_subcores)
    h1 = bag(flat_idx, P.reshape(F * VOCAB, H1))

    out = _tail(h1, gamma.reshape(1, H1), beta.reshape(1, H1), W2,
                b2.reshape(1, H2), W3.reshape(H2, 1), b3.reshape(1, 1))
    return out.reshape(B)


# speedup vs baseline: 1.4975x; 1.4975x over previous
import functools

import jax
import jax.numpy as jnp
from jax import lax
from jax.experimental import pallas as pl
from jax.experimental.pallas import tpu as pltpu
from jax.experimental.pallas import tpu_sc as plsc

B = 16384
F = 26
VOCAB = 1000
ED = 128
H1 = 128
H2 = 64
EPS = 1e-5

ROWS_PER_CHUNK = 4
IDX_PER_CHUNK = ROWS_PER_CHUNK * F
LANES = 16


def _fold_body(t_ref, w_ref, p_ref):
    p_ref[...] = jnp.dot(t_ref[0], w_ref[...],
                         preferred_element_type=jnp.float32)[None]


_fold = pl.pallas_call(
    _fold_body,
    grid=(F,),
    in_specs=[
        pl.BlockSpec((1, VOCAB, ED), lambda f: (f, 0, 0)),
        pl.BlockSpec((ED, H1), lambda f: (f, 0)),
    ],
    out_specs=pl.BlockSpec((1, VOCAB, H1), lambda f: (f, 0, 0)),
    out_shape=jax.ShapeDtypeStruct((F, VOCAB, H1), jnp.float32),
)


NBUF = 4


def _make_bag(num_cores, num_subcores):
    nw = num_cores * num_subcores
    bpw = B // nw
    nchunks = bpw // ROWS_PER_CHUNK
    mesh = plsc.VectorSubcoreMesh(core_axis_name="c", subcore_axis_name="s")

    @functools.partial(
        pl.kernel,
        mesh=mesh,
        out_type=jax.ShapeDtypeStruct((B, H1), jnp.float32),
        scratch_types=[
            pltpu.VMEM((nchunks, IDX_PER_CHUNK), jnp.int32),
        ]
        + [pltpu.VMEM((IDX_PER_CHUNK, H1), jnp.float32) for _ in range(NBUF)]
        + [pltpu.VMEM((ROWS_PER_CHUNK, H1), jnp.float32) for _ in range(NBUF)]
        + [pltpu.SemaphoreType.DMA for _ in range(2 * NBUF)],
    )
    def bag(idx_hbm, p_hbm, out_hbm, idx_v, *rest):
        g_v = rest[:NBUF]
        o_v = rest[NBUF:2 * NBUF]
        sems = rest[2 * NBUF:3 * NBUF]
        osems = rest[3 * NBUF:]
        wid = lax.axis_index("s") * num_cores + lax.axis_index("c")
        base = wid * bpw
        pltpu.sync_copy(idx_hbm.at[wid], idx_v)

        for b in range(NBUF):
            pltpu.async_copy(p_hbm.at[idx_v.at[b]], g_v[b], sems[b])

        def outer(c0, carry):
            for b in range(NBUF):
                c = c0 + b
                pltpu.make_async_copy(p_hbm.at[idx_v.at[c]],
                                      g_v[b], sems[b]).wait()

                @pl.when(c >= NBUF)
                def _():
                    pltpu.make_async_copy(
                        o_v[b], out_hbm.at[pl.ds(base, ROWS_PER_CHUNK)],
                        osems[b]).wait()

                for r in range(ROWS_PER_CHUNK):
                    for v in range(H1 // LANES):
                        sl = pl.ds(v * LANES, LANES)
                        acc0 = g_v[b][r * F, sl]
                        acc1 = g_v[b][r * F + 1, sl]
                        for f in range(2, F, 2):
                            acc0 = acc0 + g_v[b][r * F + f, sl]
                            acc1 = acc1 + g_v[b][r * F + f + 1, sl]
                        o_v[b][r, sl] = acc0 + acc1

                @pl.when(c + NBUF < nchunks)
                def _():
                    pltpu.async_copy(p_hbm.at[idx_v.at[c + NBUF]],
                                     g_v[b], sems[b])

                pltpu.async_copy(
                    o_v[b],
                    out_hbm.at[pl.ds(base + c * ROWS_PER_CHUNK,
                                     ROWS_PER_CHUNK)],
                    osems[b])
            return carry

        lax.fori_loop(0, nchunks // NBUF, lambda i, cr: outer(i * NBUF, cr),
                      0)
        for b in range(NBUF):
            pltpu.make_async_copy(
                o_v[b], out_hbm.at[pl.ds(base, ROWS_PER_CHUNK)],
                osems[b]).wait()

    return bag


_TAIL_CHUNKS = 16
_TAIL_ROWS = B // _TAIL_CHUNKS


def _tail_body(h_ref, g_ref, be_ref, w2_ref, b2_ref, w3_ref, b3_ref,
               o_ref, sum_ref, sq_ref):
    p = pl.program_id(0)
    j = pl.program_id(1)

    @pl.when(jnp.logical_and(p == 0, j == 0))
    def _():
        sum_ref[...] = jnp.zeros_like(sum_ref)
        sq_ref[...] = jnp.zeros_like(sq_ref)

    @pl.when(p == 0)
    def _():
        h = h_ref[...]
        sum_ref[...] += jnp.sum(h, axis=0, keepdims=True)
        sq_ref[...] += jnp.sum(h * h, axis=0, keepdims=True)

    @pl.when(p == 1)
    def _():
        h = h_ref[...]
        mean = sum_ref[...] * (1.0 / B)
        var = sq_ref[...] * (1.0 / B) - mean * mean
        hn = g_ref[...] * (h - mean) * lax.rsqrt(var + EPS) + be_ref[...]
        h1 = jnp.maximum(hn, 0.0)
        h2 = jnp.maximum(
            jnp.dot(h1, w2_ref[...], preferred_element_type=jnp.float32)
            + b2_ref[...], 0.0)
        o_ref[...] = (jnp.dot(h2, w3_ref[...],
                              preferred_element_type=jnp.float32)
                      + b3_ref[...])


_tail = pl.pallas_call(
    _tail_body,
    grid=(2, _TAIL_CHUNKS),
    in_specs=[
        pl.BlockSpec((_TAIL_ROWS, H1), lambda p, j: (j, 0)),
        pl.BlockSpec((1, H1), lambda p, j: (0, 0)),
        pl.BlockSpec((1, H1), lambda p, j: (0, 0)),
        pl.BlockSpec((H1, H2), lambda p, j: (0, 0)),
        pl.BlockSpec((1, H2), lambda p, j: (0, 0)),
        pl.BlockSpec((H2, 1), lambda p, j: (0, 0)),
        pl.BlockSpec((1, 1), lambda p, j: (0, 0)),
    ],
    out_specs=pl.BlockSpec((_TAIL_ROWS, 1), lambda p, j: (j, 0)),
    out_shape=jax.ShapeDtypeStruct((B, 1), jnp.float32),
    scratch_shapes=[
        pltpu.VMEM((1, H1), jnp.float32),
        pltpu.VMEM((1, H1), jnp.float32),
    ],
)


def kernel(x, tables, W1, b1, gamma, beta, W2, b2, W3, b3):
    del b1
    P = _fold(tables, W1)

    info = plsc.get_sparse_core_info()
    nw = info.num_cores * info.num_subcores
    bpw = B // nw
    nchunks = bpw // ROWS_PER_CHUNK
    flat_idx = (x.astype(jnp.int32)
                + (jnp.arange(F, dtype=jnp.int32) * VOCAB)[None, :])
    flat_idx = flat_idx.reshape(nw, nchunks, IDX_PER_CHUNK)

    bag = _make_bag(info.num_cores, info.num_subcores)
    h1 = bag(flat_idx, P.reshape(F * VOCAB, H1))

    out = _tail(h1, gamma.reshape(1, H1), beta.reshape(1, H1), W2,
                b2.reshape(1, H2), W3.reshape(H2, 1), b3.reshape(1, 1))
    return out.reshape(B)


# speedup vs baseline: 1.5559x; 1.0390x over previous
import functools

import jax
import jax.numpy as jnp
from jax import lax
from jax.experimental import pallas as pl
from jax.experimental.pallas import tpu as pltpu
from jax.experimental.pallas import tpu_sc as plsc

B = 16384
F = 26
VOCAB = 1000
ED = 128
H1 = 128
H2 = 64
EPS = 1e-5

ROWS_PER_CHUNK = 4
IDX_PER_CHUNK = ROWS_PER_CHUNK * F
LANES = 16


def _fold_body(t_ref, w_ref, p_ref):
    p_ref[...] = jnp.dot(t_ref[0], w_ref[...],
                         preferred_element_type=jnp.float32)[None]


_fold = pl.pallas_call(
    _fold_body,
    grid=(F,),
    in_specs=[
        pl.BlockSpec((1, VOCAB, ED), lambda f: (f, 0, 0)),
        pl.BlockSpec((ED, H1), lambda f: (f, 0)),
    ],
    out_specs=pl.BlockSpec((1, VOCAB, H1), lambda f: (f, 0, 0)),
    out_shape=jax.ShapeDtypeStruct((F, VOCAB, H1), jnp.float32),
)


NBUF = 4


def _make_bag(num_cores, num_subcores):
    nw = num_cores * num_subcores
    bpw = B // nw
    nchunks = bpw // ROWS_PER_CHUNK
    mesh = plsc.VectorSubcoreMesh(core_axis_name="c", subcore_axis_name="s")

    @functools.partial(
        pl.kernel,
        mesh=mesh,
        out_type=jax.ShapeDtypeStruct((B, H1), jnp.float32),
        scratch_types=[
            pltpu.VMEM((nchunks, IDX_PER_CHUNK), jnp.int32),
        ]
        + [pltpu.VMEM((IDX_PER_CHUNK, H1), jnp.float32) for _ in range(NBUF)]
        + [pltpu.VMEM((ROWS_PER_CHUNK, H1), jnp.float32) for _ in range(NBUF)]
        + [pltpu.SemaphoreType.DMA for _ in range(2 * NBUF)],
    )
    def bag(idx_hbm, p_hbm, out_hbm, idx_v, *rest):
        g_v = rest[:NBUF]
        o_v = rest[NBUF:2 * NBUF]
        sems = rest[2 * NBUF:3 * NBUF]
        osems = rest[3 * NBUF:]
        wid = lax.axis_index("s") * num_cores + lax.axis_index("c")
        base = wid * bpw
        pltpu.sync_copy(idx_hbm.at[wid], idx_v)

        for b in range(NBUF):
            pltpu.async_copy(p_hbm.at[idx_v.at[b]], g_v[b], sems[b])

        def outer(c0, carry):
            for b in range(NBUF):
                c = c0 + b
                pltpu.make_async_copy(p_hbm.at[idx_v.at[c]],
                                      g_v[b], sems[b]).wait()

                @pl.when(c >= NBUF)
                def _():
                    pltpu.make_async_copy(
                        o_v[b], out_hbm.at[pl.ds(base, ROWS_PER_CHUNK)],
                        osems[b]).wait()

                for r in range(ROWS_PER_CHUNK):
                    for v in range(H1 // LANES):
                        sl = pl.ds(v * LANES, LANES)
                        acc = g_v[b][r * F, sl]
                        for f in range(1, F):
                            acc = acc + g_v[b][r * F + f, sl]
                        o_v[b][r, sl] = acc

                @pl.when(c + NBUF < nchunks)
                def _():
                    pltpu.async_copy(p_hbm.at[idx_v.at[c + NBUF]],
                                     g_v[b], sems[b])

                pltpu.async_copy(
                    o_v[b],
                    out_hbm.at[pl.ds(base + c * ROWS_PER_CHUNK,
                                     ROWS_PER_CHUNK)],
                    osems[b])
            return carry

        lax.fori_loop(0, nchunks // NBUF, lambda i, cr: outer(i * NBUF, cr),
                      0)
        for b in range(NBUF):
            pltpu.make_async_copy(
                o_v[b], out_hbm.at[pl.ds(base, ROWS_PER_CHUNK)],
                osems[b]).wait()

    return bag


_TAIL_CHUNKS = 16
_TAIL_ROWS = B // _TAIL_CHUNKS


def _tail_body(h_ref, g_ref, be_ref, w2_ref, b2_ref, w3_ref, b3_ref,
               o_ref, sum_ref, sq_ref):
    p = pl.program_id(0)
    j = pl.program_id(1)

    @pl.when(jnp.logical_and(p == 0, j == 0))
    def _():
        sum_ref[...] = jnp.zeros_like(sum_ref)
        sq_ref[...] = jnp.zeros_like(sq_ref)

    @pl.when(p == 0)
    def _():
        h = h_ref[...]
        sum_ref[...] += jnp.sum(h, axis=0, keepdims=True)
        sq_ref[...] += jnp.sum(h * h, axis=0, keepdims=True)

    @pl.when(p == 1)
    def _():
        h = h_ref[...]
        mean = sum_ref[...] * (1.0 / B)
        var = sq_ref[...] * (1.0 / B) - mean * mean
        hn = g_ref[...] * (h - mean) * lax.rsqrt(var + EPS) + be_ref[...]
        h1 = jnp.maximum(hn, 0.0)
        h2 = jnp.maximum(
            jnp.dot(h1, w2_ref[...], preferred_element_type=jnp.float32)
            + b2_ref[...], 0.0)
        o_ref[...] = (jnp.dot(h2, w3_ref[...],
                              preferred_element_type=jnp.float32)
                      + b3_ref[...])


_tail = pl.pallas_call(
    _tail_body,
    grid=(2, _TAIL_CHUNKS),
    in_specs=[
        pl.BlockSpec((_TAIL_ROWS, H1), lambda p, j: (j, 0)),
        pl.BlockSpec((1, H1), lambda p, j: (0, 0)),
        pl.BlockSpec((1, H1), lambda p, j: (0, 0)),
        pl.BlockSpec((H1, H2), lambda p, j: (0, 0)),
        pl.BlockSpec((1, H2), lambda p, j: (0, 0)),
        pl.BlockSpec((H2, 1), lambda p, j: (0, 0)),
        pl.BlockSpec((1, 1), lambda p, j: (0, 0)),
    ],
    out_specs=pl.BlockSpec((_TAIL_ROWS, 1), lambda p, j: (j, 0)),
    out_shape=jax.ShapeDtypeStruct((B, 1), jnp.float32),
    scratch_shapes=[
        pltpu.VMEM((1, H1), jnp.float32),
        pltpu.VMEM((1, H1), jnp.float32),
    ],
)


def kernel(x, tables, W1, b1, gamma, beta, W2, b2, W3, b3):
    del b1
    P = _fold(tables, W1)

    info = plsc.get_sparse_core_info()
    nw = info.num_cores * info.num_subcores
    bpw = B // nw
    nchunks = bpw // ROWS_PER_CHUNK
    flat_idx = (x.astype(jnp.int32)
                + (jnp.arange(F, dtype=jnp.int32) * VOCAB)[None, :])
    flat_idx = flat_idx.reshape(nw, nchunks, IDX_PER_CHUNK)

    bag = _make_bag(info.num_cores, info.num_subcores)
    h1 = bag(flat_idx, P.reshape(F * VOCAB, H1))

    out = _tail(h1, gamma.reshape(1, H1), beta.reshape(1, H1), W2,
                b2.reshape(1, H2), W3.reshape(H2, 1), b3.reshape(1, 1))
    return out.reshape(B)
